# baseline (speedup 1.0000x reference)
import functools

import jax
import jax.numpy as jnp
from jax import lax
from jax.experimental import pallas as pl
from jax.experimental.pallas import tpu as pltpu
from jax.experimental.pallas import tpu_sc as plsc

B, U, T = 4096, 2048, 1024
NC, NS, L = 2, 16, 16
NW = NC * NS
RPW = B // NW
NBIN = T + L
WN = T * U
WP = WN // 2


def kernel(x, w, b):
    w16 = w.astype(jnp.bfloat16).reshape(-1, 2)
    u16 = jax.lax.bitcast_convert_type(w16, jnp.uint16)
    wpair = (u16[:, 0].astype(jnp.int32)
             | (u16[:, 1].astype(jnp.int32) << 16))
    mesh = plsc.VectorSubcoreMesh(core_axis_name="c", subcore_axis_name="s")

    @functools.partial(
        pl.kernel,
        mesh=mesh,
        out_type=jax.ShapeDtypeStruct((B, T), jnp.float32),
        compiler_params=pltpu.CompilerParams(
            needs_layout_passes=False, use_tc_tiling_on_sc=False),
        scratch_types=[
            pltpu.VMEM((U,), jnp.int32),
            pltpu.VMEM((U,), jnp.int32),
            pltpu.VMEM((U,), jnp.int32),
            pltpu.VMEM((U,), jnp.int32),
            pltpu.VMEM((U,), jnp.int32),
            pltpu.VMEM((U,), jnp.int32),
            pltpu.VMEM((NBIN,), jnp.float32),
            pltpu.VMEM((NBIN,), jnp.float32),
            pltpu.VMEM((T,), jnp.float32),
            pltpu.VMEM_SHARED((WP,), jnp.int32),
            pltpu.SemaphoreType.DMA,
            pltpu.SemaphoreType.DMA,
            pltpu.SemaphoreType.DMA,
            pltpu.SemaphoreType.DMA,
            pltpu.SemaphoreType.DMA,
            pltpu.SemaphoreType.DMA,
        ],
    )
    def sck(x_hbm, w_hbm, b_hbm, out_hbm,
            xrow0, xrow1, gidx0, gidx1, gval0, gval1, acc0, acc1, bias,
            spw, sx0, sx1, sg0, sg1, so0, so1):
        xrow = (xrow0, xrow1)
        gidx = (gidx0, gidx1)
        gval = (gval0, gval1)
        acc = (acc0, acc1)
        sx = (sx0, sx1)
        sg = (sg0, sg1)
        so = (so0, so1)
        sid = lax.axis_index("s")
        wid = sid * NC + lax.axis_index("c")
        row0 = wid * RPW
        last_row = row0 + RPW - 1
        pltpu.sync_copy(b_hbm, bias)
        col = lax.iota(jnp.int32, L)
        odd = (col & 1) == 1

        chunk = WP // NS
        pltpu.sync_copy(w_hbm.at[pl.ds(sid * chunk, chunk)],
                        spw.at[pl.ds(sid * chunk, chunk)])
        plsc.subcore_barrier()

        def compute_idx(xr, gi):
            @plsc.parallel_loop(0, U // L, unroll=8)
            def _(i):
                xv = xr[pl.ds(i * L, L)]
                flat = jnp.maximum(xv * U + (col + (i * L - U)), 0)
                gi[pl.ds(i * L, L)] = lax.shift_right_logical(flat, 1)

        def init_acc(a):
            @plsc.parallel_loop(0, T // L, unroll=8)
            def _(j):
                a[pl.ds(j * L + L, L)] = bias[pl.ds(j * L, L)]

        def scatter_row(xr, gv, a):
            def si(i, c):
                pv = gv[pl.ds(i * L, L)]
                lo = lax.shift_left(pv, 16)
                hi = pv & jnp.int32(-65536)
                vv = plsc.bitcast(jnp.where(odd, hi, lo), jnp.float32)
                xv = xr[pl.ds(i * L, L)]
                plsc.addupdate_scatter(a, [xv + (L - 1)], vv)
                return c
            lax.fori_loop(0, U // L, si, 0, unroll=8)

        def handle(i, p, first_pair):
            q = 1 - p
            row = row0 + i
            pltpu.make_async_copy(x_hbm.at[row], xrow[p], sx[p]).wait()
            compute_idx(xrow[p], gidx[p])
            pltpu.async_copy(spw.at[gidx[p]], gval[p], sg[p])
            if not first_pair:
                pltpu.make_async_copy(
                    acc[p].at[pl.ds(L, T)], out_hbm.at[row], so[p]).wait()
            init_acc(acc[p])
            pltpu.make_async_copy(spw.at[gidx[q]], gval[q], sg[q]).wait()
            scatter_row(xrow[q], gval[q], acc[q])
            pltpu.async_copy(
                acc[q].at[pl.ds(L, T)], out_hbm.at[row - 1], so[q])
            nxt = jnp.minimum(row + 1, last_row)
            pltpu.async_copy(x_hbm.at[nxt], xrow[q], sx[q])

        pltpu.sync_copy(x_hbm.at[row0], xrow0)
        compute_idx(xrow0, gidx0)
        pltpu.async_copy(spw.at[gidx0], gval0, sg0)
        pltpu.async_copy(x_hbm.at[row0 + 1], xrow1, sx1)
        init_acc(acc0)
        handle(1, 1, True)

        def pair_body(j, c):
            handle(2 * j, 0, False)
            handle(2 * j + 1, 1, False)
            return c
        lax.fori_loop(1, RPW // 2, pair_body, 0)

        pltpu.make_async_copy(spw.at[gidx1], gval1, sg1).wait()
        scatter_row(xrow1, gval1, acc1)
        pltpu.sync_copy(acc1.at[pl.ds(L, T)], out_hbm.at[last_row])
        pltpu.make_async_copy(
            acc0.at[pl.ds(L, T)], out_hbm.at[last_row], so0).wait()
        pltpu.make_async_copy(x_hbm.at[last_row], xrow0, sx0).wait()

    return sck(x, wpair, b)


# speedup vs baseline: 2.9072x; 2.9072x over previous
import functools

import jax
import jax.numpy as jnp
from jax import lax
from jax.experimental import pallas as pl
from jax.experimental.pallas import tpu as pltpu
from jax.experimental.pallas import tpu_sc as plsc

B, U, T = 4096, 2048, 1024
NC, NS, L = 2, 16, 16
NW = NC * NS
RPW = B // NW
NBIN = T + L

ABLATE_GATHER = False
ABLATE_SCATTER = False


def kernel(x, w, b):
    w_flat = w.reshape(-1)
    mesh = plsc.VectorSubcoreMesh(core_axis_name="c", subcore_axis_name="s")

    @functools.partial(
        pl.kernel,
        mesh=mesh,
        out_type=jax.ShapeDtypeStruct((B, T), jnp.float32),
        compiler_params=pltpu.CompilerParams(
            needs_layout_passes=False, use_tc_tiling_on_sc=False),
        scratch_types=[
            pltpu.VMEM((U,), jnp.int32),
            pltpu.VMEM((U,), jnp.int32),
            pltpu.VMEM((U,), jnp.int32),
            pltpu.VMEM((U,), jnp.int32),
            pltpu.VMEM((U,), jnp.float32),
            pltpu.VMEM((U,), jnp.float32),
            pltpu.VMEM((NBIN,), jnp.float32),
            pltpu.VMEM((NBIN,), jnp.float32),
            pltpu.VMEM((T,), jnp.float32),
            pltpu.SemaphoreType.DMA,
            pltpu.SemaphoreType.DMA,
            pltpu.SemaphoreType.DMA,
            pltpu.SemaphoreType.DMA,
            pltpu.SemaphoreType.DMA,
            pltpu.SemaphoreType.DMA,
        ],
    )
    def sck(x_hbm, w_hbm, b_hbm, out_hbm,
            xrow0, xrow1, gidx0, gidx1, gval0, gval1, acc0, acc1, bias,
            sx0, sx1, sg0, sg1, so0, so1):
        xrow = (xrow0, xrow1)
        gidx = (gidx0, gidx1)
        gval = (gval0, gval1)
        acc = (acc0, acc1)
        sx = (sx0, sx1)
        sg = (sg0, sg1)
        so = (so0, so1)
        wid = lax.axis_index("s") * NC + lax.axis_index("c")
        row0 = wid * RPW
        last_row = row0 + RPW - 1
        pltpu.sync_copy(b_hbm, bias)
        col = lax.iota(jnp.int32, L)

        def start_gather(p):
            if ABLATE_GATHER:
                pltpu.async_copy(w_hbm.at[pl.ds(0, U)], gval[p], sg[p])
            else:
                pltpu.async_copy(w_hbm.at[gidx[p]], gval[p], sg[p])

        def wait_gather(q):
            if ABLATE_GATHER:
                pltpu.make_async_copy(
                    w_hbm.at[pl.ds(0, U)], gval[q], sg[q]).wait()
            else:
                pltpu.make_async_copy(w_hbm.at[gidx[q]], gval[q], sg[q]).wait()

        def compute_idx(xr, gi):
            @plsc.parallel_loop(0, U // L, unroll=8)
            def _(i):
                xv = xr[pl.ds(i * L, L)]
                flat = xv * U + (col + (i * L - U))
                gi[pl.ds(i * L, L)] = jnp.maximum(flat, 0)

        def init_acc(a):
            @plsc.parallel_loop(0, T // L, unroll=8)
            def _(j):
                a[pl.ds(j * L + L, L)] = bias[pl.ds(j * L, L)]

        def scatter_row(xr, gv, a):
            if ABLATE_SCATTER:
                return
            def si(i, c):
                xv = xr[pl.ds(i * L, L)]
                vv = gv[pl.ds(i * L, L)]
                plsc.addupdate_scatter(a, [xv + (L - 1)], vv)
                return c
            lax.fori_loop(0, U // L, si, 0, unroll=8)

        def handle(i, p, first_pair):
            q = 1 - p
            row = row0 + i
            pltpu.make_async_copy(x_hbm.at[row], xrow[p], sx[p]).wait()
            compute_idx(xrow[p], gidx[p])
            start_gather(p)
            if not first_pair:
                pltpu.make_async_copy(
                    acc[p].at[pl.ds(L, T)], out_hbm.at[row], so[p]).wait()
            init_acc(acc[p])
            wait_gather(q)
            scatter_row(xrow[q], gval[q], acc[q])
            pltpu.async_copy(
                acc[q].at[pl.ds(L, T)], out_hbm.at[row - 1], so[q])
            nxt = jnp.minimum(row + 1, last_row)
            pltpu.async_copy(x_hbm.at[nxt], xrow[q], sx[q])

        pltpu.sync_copy(x_hbm.at[row0], xrow0)
        compute_idx(xrow0, gidx0)
        start_gather(0)
        pltpu.async_copy(x_hbm.at[row0 + 1], xrow1, sx1)
        init_acc(acc0)
        handle(1, 1, True)

        def pair_body(j, c):
            handle(2 * j, 0, False)
            handle(2 * j + 1, 1, False)
            return c
        lax.fori_loop(1, RPW // 2, pair_body, 0)

        wait_gather(1)
        scatter_row(xrow1, gval1, acc1)
        pltpu.sync_copy(acc1.at[pl.ds(L, T)], out_hbm.at[last_row])
        pltpu.make_async_copy(
            acc0.at[pl.ds(L, T)], out_hbm.at[last_row], so0).wait()
        pltpu.make_async_copy(x_hbm.at[last_row], xrow0, sx0).wait()

    return sck(x, w_flat, b)
